# baseline (speedup 1.0000x reference)
import jax
import jax.numpy as jnp
from jax import lax
from jax.experimental import pallas as pl
from jax.experimental.pallas import tpu as pltpu
from jax.experimental.pallas import tpu_sc as plsc

N = 8192
K = 32
CUTOFF2 = 25.0
BIG = 1e18
BIGHALF = 5e17
NC = 2
NS = 16
NW = NC * NS
ROWS_PER = N // NW
L = 16
NSEG = 64
CH = 512
WL = N + CH + 32


def _rev(v):
  return lax.rev(v, (0,))


def _sload(ref, idx):
  return ref[pl.ds(idx, L)][0]


def _merge_low(ak, av, bk, bv):
  rbk, rbv = _rev(bk), _rev(bv)
  take_a = ak <= rbk
  lk = jnp.where(take_a, ak, rbk)
  lv = jnp.where(take_a, av, rbv)
  return plsc.sort_key_val(lk, lv)


def _merge_both(ak, av, bk, bv):
  rbk, rbv = _rev(bk), _rev(bv)
  take_a = ak <= rbk
  lk = jnp.where(take_a, ak, rbk)
  lv = jnp.where(take_a, av, rbv)
  hk = jnp.where(take_a, rbk, ak)
  hv = jnp.where(take_a, rbv, av)
  lk, lv = plsc.sort_key_val(lk, lv)
  hk, hv = plsc.sort_key_val(hk, hv)
  return lk, lv, hk, hv


def _bf16_rne(x):
  i = plsc.bitcast(x, jnp.int32)
  r = (i + 0x7FFF + (lax.shift_right_logical(i, 16) & 1)) & jnp.int32(-65536)
  return plsc.bitcast(r, jnp.float32)


def _sqrt(a):
  i = plsc.bitcast(a, jnp.int32)
  y = plsc.bitcast(jnp.int32(0x1FBD1DF5) + lax.shift_right_logical(i, 1),
                   jnp.float32)
  for _ in range(3):
    y = 0.5 * (y + a / y)
  return jnp.where(a > 0.0, y, 0.0)


def _body(x_hbm, y_hbm, z_hbm, b_hbm, s_hbm,
          ei_hbm, w_hbm, vec_hbm,
          xv, yv, zv, xbv, ybv, zbv, x2v, bvw, sv,
          srcb, tgtb, wb, vecb, csem):
  wid = lax.axis_index("s") * NC + lax.axis_index("c")
  row0 = wid * ROWS_PER

  with jax.named_scope("stage"):
    pltpu.sync_copy(b_hbm.at[pl.ds(row0, ROWS_PER)], bvw.at[pl.ds(0, ROWS_PER)])
    pltpu.sync_copy(s_hbm, sv)

  bid_first = _sload(bvw, 0)
  bid_last = _sload(bvw, ROWS_PER - 1)
  base = sv[pl.ds(bid_first, L)][0]
  end = sv[pl.ds(bid_last, L)][1]
  base_al = pl.multiple_of(base & -8, 8)
  span = end - base_al

  def copy_body(j, _):
    o = j * CH
    h = pltpu.async_copy(x_hbm.at[pl.ds(base_al + o, CH)],
                         xv.at[pl.ds(o, CH)], csem)
    h2 = pltpu.async_copy(y_hbm.at[pl.ds(base_al + o, CH)],
                          yv.at[pl.ds(o, CH)], csem)
    h3 = pltpu.async_copy(z_hbm.at[pl.ds(base_al + o, CH)],
                          zv.at[pl.ds(o, CH)], csem)
    h.wait()
    h2.wait()
    h3.wait()
    return _

  with jax.named_scope("copyin"):
    lax.fori_loop(0, (span + L + CH - 1) // CH, copy_body, None)

  def pre_body(j, _):
    o = j * L
    cx = xv[pl.ds(o, L)]
    cy = yv[pl.ds(o, L)]
    cz = zv[pl.ds(o, L)]
    xbv[pl.ds(o, L)] = _bf16_rne(cx)
    ybv[pl.ds(o, L)] = _bf16_rne(cy)
    zbv[pl.ds(o, L)] = _bf16_rne(cz)
    x2v[pl.ds(o, L)] = cx * cx + cy * cy + cz * cz
    return _

  with jax.named_scope("pre"):
    lax.fori_loop(0, span // L + 1, pre_body, None)

  lanes = lax.iota(jnp.int32, L)

  def row_body(rp, _):
    i0 = row0 + rp * 2
    i1 = i0 + 1
    l0 = i0 - base_al
    l1 = i1 - base_al
    bid0 = _sload(bvw, rp * 2)
    bid1 = _sload(bvw, rp * 2 + 1)
    se0 = sv[pl.ds(bid0, L)]
    se1 = sv[pl.ds(bid1, L)]
    s0, e0 = se0[0], se0[1]
    s1, e1 = se1[0], se1[1]
    q = [(i0, l0, _sload(xbv, l0), _sload(ybv, l0), _sload(zbv, l0),
          _sload(x2v, l0), s0, e0),
         (i1, l1, _sload(xbv, l1), _sload(ybv, l1), _sload(zbv, l1),
          _sload(x2v, l1), s1, e1)]
    nchunks = (e1 - s0 + 2 * L - 1) // (2 * L)

    def chunk_body(c, carry):
      off = s0 + c * (2 * L)
      lo = off - base_al
      idxa = off + lanes
      idxb = idxa + L
      cxa = xbv[pl.ds(lo, L)]
      cya = ybv[pl.ds(lo, L)]
      cza = zbv[pl.ds(lo, L)]
      c2a = x2v[pl.ds(lo, L)]
      cxb = xbv[pl.ds(lo + L, L)]
      cyb = ybv[pl.ds(lo + L, L)]
      czb = zbv[pl.ds(lo + L, L)]
      c2b = x2v[pl.ds(lo + L, L)]
      out = []
      for r, (i, il, qxb, qyb, qzb, qx2, s, e) in enumerate(q):
        t0k, t0v, t1k, t1v = carry[4 * r:4 * r + 4]
        dota = cxa * qxb + cya * qyb + cza * qzb
        d2a = jnp.maximum(qx2 + c2a - 2.0 * dota, 0.0)
        oka = (idxa >= s) & (idxa < e) & (d2a <= CUTOFF2)
        dotb = cxb * qxb + cyb * qyb + czb * qzb
        d2b = jnp.maximum(qx2 + c2b - 2.0 * dotb, 0.0)
        okb = (idxb >= s) & (idxb < e) & (d2b <= CUTOFF2)
        ak, av = plsc.sort_key_val(jnp.where(oka, d2a, BIG), idxa)
        bk, bv_ = plsc.sort_key_val(jnp.where(okb, d2b, BIG), idxb)
        n0k, n0v, n1k, n1v = _merge_both(ak, av, bk, bv_)
        rk, rv = _rev(n1k), _rev(n1v)
        m0 = t0k <= rk
        l0k = jnp.where(m0, t0k, rk)
        l0v = jnp.where(m0, t0v, rv)
        rk, rv = _rev(n0k), _rev(n0v)
        m1 = t1k <= rk
        l1k = jnp.where(m1, t1k, rk)
        l1v = jnp.where(m1, t1v, rv)
        mp = l0k <= l1k
        pk = jnp.where(mp, l0k, l1k)
        pv = jnp.where(mp, l0v, l1v)
        qk = jnp.where(mp, l1k, l0k)
        qv = jnp.where(mp, l1v, l0v)
        t0k, t0v = plsc.sort_key_val(pk, pv)
        t1k, t1v = plsc.sort_key_val(qk, qv)
        out += [t0k, t0v, t1k, t1v]
      return tuple(out)

    def full16(val, dtype):
      return jnp.full((L,), val, dtype=dtype)

    init = (full16(BIG, jnp.float32), full16(i0, jnp.int32),
            full16(BIG, jnp.float32), full16(i0, jnp.int32),
            full16(BIG, jnp.float32), full16(i1, jnp.int32),
            full16(BIG, jnp.float32), full16(i1, jnp.int32))
    res = lax.fori_loop(0, nchunks, chunk_body, init)

    for r, (i, il, qxb, qyb, qzb, qx2, s, e) in enumerate(q):
      qx, qy, qz = _sload(xv, il), _sload(yv, il), _sload(zv, il)
      ro = (rp * 2 + r) * K
      for half in range(2):
        tk, tv = res[4 * r + 2 * half], res[4 * r + 2 * half + 1]
        src = jnp.where(tk < BIGHALF, tv, i)
        srcl = src - base_al
        dx = plsc.load_gather(xv, [srcl]) - qx
        dy = plsc.load_gather(yv, [srcl]) - qy
        dz = plsc.load_gather(zv, [srcl]) - qz
        o = ro + half * L
        srcb[pl.ds(o, L)] = src
        tgtb[pl.ds(o, L)] = jnp.full((L,), i, dtype=jnp.int32)
        wb[pl.ds(o, L)] = _sqrt(dx * dx + dy * dy + dz * dz)
        flat3 = (o + lanes) * 3
        plsc.store_scatter(vecb, [flat3], dx)
        plsc.store_scatter(vecb, [flat3 + 1], dy)
        plsc.store_scatter(vecb, [flat3 + 2], dz)
    return _

  with jax.named_scope("rows"):
    lax.fori_loop(0, ROWS_PER // 2, row_body, None)

  eb = row0 * K
  ne = ROWS_PER * K
  with jax.named_scope("out"):
    handles = [
      pltpu.async_copy(srcb, ei_hbm.at[pl.ds(eb, ne)], csem),
      pltpu.async_copy(tgtb, ei_hbm.at[pl.ds(N * K + eb, ne)], csem),
      pltpu.async_copy(wb, w_hbm.at[pl.ds(eb, ne)], csem),
      pltpu.async_copy(vecb, vec_hbm.at[pl.ds(3 * eb, 3 * ne)], csem),
    ]
    for h in handles:
      h.wait()


@jax.jit
def kernel(pos, batch):
  batch32 = batch.astype(jnp.int32)
  hpad = jnp.zeros((CH + 16,), dtype=jnp.float32)
  x = jnp.concatenate([pos[:, 0], hpad])
  y = jnp.concatenate([pos[:, 1], hpad])
  z = jnp.concatenate([pos[:, 2], hpad])
  segs = jnp.sum(
      batch32[None, :] < jnp.arange(NSEG + 24, dtype=jnp.int32)[:, None],
      axis=1, dtype=jnp.int32)

  E = N * K
  f32 = jnp.float32
  outs = pl.kernel(
      _body,
      out_type=[
          jax.ShapeDtypeStruct((2 * E,), jnp.int32),
          jax.ShapeDtypeStruct((E,), f32),
          jax.ShapeDtypeStruct((3 * E,), f32),
      ],
      mesh=plsc.VectorSubcoreMesh(core_axis_name="c", subcore_axis_name="s",
                                  num_cores=NC, num_subcores=NS),
      compiler_params=pltpu.CompilerParams(needs_layout_passes=False),
      scratch_types=[
          pltpu.VMEM((WL,), f32),
          pltpu.VMEM((WL,), f32),
          pltpu.VMEM((WL,), f32),
          pltpu.VMEM((WL,), f32),
          pltpu.VMEM((WL,), f32),
          pltpu.VMEM((WL,), f32),
          pltpu.VMEM((WL,), f32),
          pltpu.VMEM((ROWS_PER + L,), jnp.int32),
          pltpu.VMEM((NSEG + 24,), jnp.int32),
          pltpu.VMEM((ROWS_PER * K,), jnp.int32),
          pltpu.VMEM((ROWS_PER * K,), jnp.int32),
          pltpu.VMEM((ROWS_PER * K,), f32),
          pltpu.VMEM((ROWS_PER * K * 3,), f32),
          pltpu.SemaphoreType.DMA,
      ],
  )(x, y, z, batch32, segs)

  ei_flat, w, vec_flat = outs
  return ei_flat.reshape(2, E), w, vec_flat.reshape(E, 3)


# speedup vs baseline: 3.3982x; 3.3982x over previous
import jax
import jax.numpy as jnp
from jax import lax
from jax.experimental import pallas as pl
from jax.experimental.pallas import tpu as pltpu
from jax.experimental.pallas import tpu_sc as plsc

N = 8192
K = 32
CUTOFF2 = 25.0
BIG = 1e18
BIGHALF = 5e17
NC = 2
NS = 16
NW = NC * NS
ROWS_PER = N // NW
L = 16
NSEG = 64
CH = 512
WL = N + CH + 32


def _rev(v):
  return lax.rev(v, (0,))


def _sload(ref, idx):
  return ref[pl.ds(idx, L)][0]


def _merge_low(ak, av, bk, bv):
  rbk, rbv = _rev(bk), _rev(bv)
  take_a = ak <= rbk
  lk = jnp.where(take_a, ak, rbk)
  lv = jnp.where(take_a, av, rbv)
  return plsc.sort_key_val(lk, lv)


def _merge_both(ak, av, bk, bv):
  rbk, rbv = _rev(bk), _rev(bv)
  take_a = ak <= rbk
  lk = jnp.where(take_a, ak, rbk)
  lv = jnp.where(take_a, av, rbv)
  hk = jnp.where(take_a, rbk, ak)
  hv = jnp.where(take_a, rbv, av)
  lk, lv = plsc.sort_key_val(lk, lv)
  hk, hv = plsc.sort_key_val(hk, hv)
  return lk, lv, hk, hv


def _bf16_rne(x):
  i = plsc.bitcast(x, jnp.int32)
  r = (i + 0x7FFF + (lax.shift_right_logical(i, 16) & 1)) & jnp.int32(-65536)
  return plsc.bitcast(r, jnp.float32)


def _sqrt(a):
  i = plsc.bitcast(a, jnp.int32)
  y = plsc.bitcast(jnp.int32(0x1FBD1DF5) + lax.shift_right_logical(i, 1),
                   jnp.float32)
  for _ in range(3):
    y = 0.5 * (y + a / y)
  return jnp.where(a > 0.0, y, 0.0)


def _body(x_hbm, y_hbm, z_hbm, b_hbm, s_hbm,
          src_hbm, tgt_hbm, w_hbm, vx_hbm, vy_hbm, vz_hbm,
          xv, yv, zv, xbv, ybv, zbv, x2v, bvw, sv,
          srcb, tgtb, wb, vxb, vyb, vzb, csem):
  wid = lax.axis_index("s") * NC + lax.axis_index("c")
  row0 = wid * ROWS_PER

  with jax.named_scope("stage"):
    pltpu.sync_copy(b_hbm.at[pl.ds(row0, ROWS_PER)], bvw.at[pl.ds(0, ROWS_PER)])
    pltpu.sync_copy(s_hbm, sv)

  bid_first = _sload(bvw, 0)
  bid_last = _sload(bvw, ROWS_PER - 1)
  base = sv[pl.ds(bid_first, L)][0]
  end = sv[pl.ds(bid_last, L)][1]
  base_al = pl.multiple_of(base & -8, 8)
  span = end - base_al

  def copy_body(j, _):
    o = j * CH
    h = pltpu.async_copy(x_hbm.at[pl.ds(base_al + o, CH)],
                         xv.at[pl.ds(o, CH)], csem)
    h2 = pltpu.async_copy(y_hbm.at[pl.ds(base_al + o, CH)],
                          yv.at[pl.ds(o, CH)], csem)
    h3 = pltpu.async_copy(z_hbm.at[pl.ds(base_al + o, CH)],
                          zv.at[pl.ds(o, CH)], csem)
    h.wait()
    h2.wait()
    h3.wait()
    return _

  with jax.named_scope("copyin"):
    lax.fori_loop(0, (span + L + CH - 1) // CH, copy_body, None)

  def pre_body(j, _):
    o = j * L
    cx = xv[pl.ds(o, L)]
    cy = yv[pl.ds(o, L)]
    cz = zv[pl.ds(o, L)]
    xbv[pl.ds(o, L)] = _bf16_rne(cx)
    ybv[pl.ds(o, L)] = _bf16_rne(cy)
    zbv[pl.ds(o, L)] = _bf16_rne(cz)
    x2v[pl.ds(o, L)] = cx * cx + cy * cy + cz * cz
    return _

  with jax.named_scope("pre"):
    lax.fori_loop(0, span // L + 1, pre_body, None)

  lanes = lax.iota(jnp.int32, L)

  def row_body(rp, _):
    i0 = row0 + rp * 2
    i1 = i0 + 1
    l0 = i0 - base_al
    l1 = i1 - base_al
    bid0 = _sload(bvw, rp * 2)
    bid1 = _sload(bvw, rp * 2 + 1)
    se0 = sv[pl.ds(bid0, L)]
    se1 = sv[pl.ds(bid1, L)]
    s0, e0 = se0[0], se0[1]
    s1, e1 = se1[0], se1[1]
    q = [(i0, l0, _sload(xbv, l0), _sload(ybv, l0), _sload(zbv, l0),
          _sload(x2v, l0), s0, e0),
         (i1, l1, _sload(xbv, l1), _sload(ybv, l1), _sload(zbv, l1),
          _sload(x2v, l1), s1, e1)]
    nchunks = (e1 - s0 + 2 * L - 1) // (2 * L)

    def chunk_body(c, carry):
      off = s0 + c * (2 * L)
      lo = off - base_al
      idxa = off + lanes
      idxb = idxa + L
      cxa = xbv[pl.ds(lo, L)]
      cya = ybv[pl.ds(lo, L)]
      cza = zbv[pl.ds(lo, L)]
      c2a = x2v[pl.ds(lo, L)]
      cxb = xbv[pl.ds(lo + L, L)]
      cyb = ybv[pl.ds(lo + L, L)]
      czb = zbv[pl.ds(lo + L, L)]
      c2b = x2v[pl.ds(lo + L, L)]
      out = []
      for r, (i, il, qxb, qyb, qzb, qx2, s, e) in enumerate(q):
        t0k, t0v, t1k, t1v = carry[4 * r:4 * r + 4]
        dota = cxa * qxb + cya * qyb + cza * qzb
        d2a = jnp.maximum(qx2 + c2a - 2.0 * dota, 0.0)
        oka = (idxa >= s) & (idxa < e) & (d2a <= CUTOFF2)
        dotb = cxb * qxb + cyb * qyb + czb * qzb
        d2b = jnp.maximum(qx2 + c2b - 2.0 * dotb, 0.0)
        okb = (idxb >= s) & (idxb < e) & (d2b <= CUTOFF2)
        ak, av = plsc.sort_key_val(jnp.where(oka, d2a, BIG), idxa)
        bk, bv_ = plsc.sort_key_val(jnp.where(okb, d2b, BIG), idxb)
        n0k, n0v, n1k, n1v = _merge_both(ak, av, bk, bv_)
        rk, rv = _rev(n1k), _rev(n1v)
        m0 = t0k <= rk
        l0k = jnp.where(m0, t0k, rk)
        l0v = jnp.where(m0, t0v, rv)
        rk, rv = _rev(n0k), _rev(n0v)
        m1 = t1k <= rk
        l1k = jnp.where(m1, t1k, rk)
        l1v = jnp.where(m1, t1v, rv)
        mp = l0k <= l1k
        pk = jnp.where(mp, l0k, l1k)
        pv = jnp.where(mp, l0v, l1v)
        qk = jnp.where(mp, l1k, l0k)
        qv = jnp.where(mp, l1v, l0v)
        t0k, t0v = plsc.sort_key_val(pk, pv)
        t1k, t1v = plsc.sort_key_val(qk, qv)
        out += [t0k, t0v, t1k, t1v]
      return tuple(out)

    def full16(val, dtype):
      return jnp.full((L,), val, dtype=dtype)

    init = (full16(BIG, jnp.float32), full16(i0, jnp.int32),
            full16(BIG, jnp.float32), full16(i0, jnp.int32),
            full16(BIG, jnp.float32), full16(i1, jnp.int32),
            full16(BIG, jnp.float32), full16(i1, jnp.int32))
    res = lax.fori_loop(0, nchunks, chunk_body, init)

    for r, (i, il, qxb, qyb, qzb, qx2, s, e) in enumerate(q):
      qx, qy, qz = _sload(xv, il), _sload(yv, il), _sload(zv, il)
      ro = (rp * 2 + r) * K
      for half in range(2):
        tk, tv = res[4 * r + 2 * half], res[4 * r + 2 * half + 1]
        src = jnp.where(tk < BIGHALF, tv, i)
        srcl = src - base_al
        dx = plsc.load_gather(xv, [srcl]) - qx
        dy = plsc.load_gather(yv, [srcl]) - qy
        dz = plsc.load_gather(zv, [srcl]) - qz
        o = ro + half * L
        srcb[pl.ds(o, L)] = src
        tgtb[pl.ds(o, L)] = jnp.full((L,), i, dtype=jnp.int32)
        wb[pl.ds(o, L)] = _sqrt(dx * dx + dy * dy + dz * dz)
        vxb[pl.ds(o, L)] = dx
        vyb[pl.ds(o, L)] = dy
        vzb[pl.ds(o, L)] = dz
    return _

  with jax.named_scope("rows"):
    lax.fori_loop(0, ROWS_PER // 2, row_body, None)

  eb = row0 * K
  ne = ROWS_PER * K
  with jax.named_scope("out"):
    handles = [
      pltpu.async_copy(srcb, src_hbm.at[pl.ds(eb, ne)], csem),
      pltpu.async_copy(tgtb, tgt_hbm.at[pl.ds(eb, ne)], csem),
      pltpu.async_copy(wb, w_hbm.at[pl.ds(eb, ne)], csem),
      pltpu.async_copy(vxb, vx_hbm.at[pl.ds(eb, ne)], csem),
      pltpu.async_copy(vyb, vy_hbm.at[pl.ds(eb, ne)], csem),
      pltpu.async_copy(vzb, vz_hbm.at[pl.ds(eb, ne)], csem),
    ]
    for h in handles:
      h.wait()


@jax.jit
def kernel(pos, batch):
  batch32 = batch.astype(jnp.int32)
  hpad = jnp.zeros((CH + 16,), dtype=jnp.float32)
  x = jnp.concatenate([pos[:, 0], hpad])
  y = jnp.concatenate([pos[:, 1], hpad])
  z = jnp.concatenate([pos[:, 2], hpad])
  segs = jnp.sum(
      batch32[None, :] < jnp.arange(NSEG + 24, dtype=jnp.int32)[:, None],
      axis=1, dtype=jnp.int32)

  E = N * K
  f32 = jnp.float32
  outs = pl.kernel(
      _body,
      out_type=[
          jax.ShapeDtypeStruct((E,), jnp.int32),
          jax.ShapeDtypeStruct((E,), jnp.int32),
          jax.ShapeDtypeStruct((E,), f32),
          jax.ShapeDtypeStruct((E,), f32),
          jax.ShapeDtypeStruct((E,), f32),
          jax.ShapeDtypeStruct((E,), f32),
      ],
      mesh=plsc.VectorSubcoreMesh(core_axis_name="c", subcore_axis_name="s",
                                  num_cores=NC, num_subcores=NS),
      compiler_params=pltpu.CompilerParams(needs_layout_passes=False),
      scratch_types=[
          pltpu.VMEM((WL,), f32),
          pltpu.VMEM((WL,), f32),
          pltpu.VMEM((WL,), f32),
          pltpu.VMEM((WL,), f32),
          pltpu.VMEM((WL,), f32),
          pltpu.VMEM((WL,), f32),
          pltpu.VMEM((WL,), f32),
          pltpu.VMEM((ROWS_PER + L,), jnp.int32),
          pltpu.VMEM((NSEG + 24,), jnp.int32),
          pltpu.VMEM((ROWS_PER * K,), jnp.int32),
          pltpu.VMEM((ROWS_PER * K,), jnp.int32),
          pltpu.VMEM((ROWS_PER * K,), f32),
          pltpu.VMEM((ROWS_PER * K,), f32),
          pltpu.VMEM((ROWS_PER * K,), f32),
          pltpu.VMEM((ROWS_PER * K,), f32),
          pltpu.SemaphoreType.DMA,
      ],
  )(x, y, z, batch32, segs)

  src, tgt, w, vx, vy, vz = outs
  edge_index = jnp.stack([src, tgt], axis=0)
  edge_vec = jnp.stack([vx, vy, vz], axis=1)
  return edge_index, w, edge_vec
